# baseline (speedup 1.0000x reference)
import functools

import jax
import jax.numpy as jnp
from jax import lax
from jax.experimental import pallas as pl
from jax.experimental.pallas import tpu as pltpu
from jax.experimental.pallas import tpu_sc as plsc

N_FIELDS = 26
N_WIDE = 6
B = 16384
VOCAB = 100000
EMB = 16
WIDE_DIM = N_WIDE * EMB
DEEP_DIM = (N_FIELDS - N_WIDE) * EMB
FDIM = N_FIELDS * EMB
H1, H2 = 256, 128

_NC, _NS = 2, 16
NW = _NC * _NS
BPW = B // NW
CHUNK = 128
NCHUNK = BPW // CHUNK

TCOLS = 782
VPAD = TCOLS * 128
N_UNITS = N_FIELDS * TCOLS
UPW = -(-N_UNITS // NW) // 2 * 2


def _sc_pack_body(tt_hbm, out_hbm, inb, outb, isem0, isem1, osem0, osem1):
    wid = lax.axis_index("s") * _NC + lax.axis_index("c")
    u0 = wid * UPW
    iota16 = lax.iota(jnp.int32, 16)
    isems = (isem0, isem1)
    osems = (osem0, osem1)

    def fire_in(u, b):
        f = u // TCOLS
        tc = u % TCOLS
        pltpu.async_copy(tt_hbm.at[pl.ds(f, 1), :, pl.ds(tc * 128, 128)],
                         inb.at[b], isems[b])

    def wait_in(b):
        pltpu.make_async_copy(tt_hbm.at[pl.ds(0, 1), :, pl.ds(0, 128)],
                              inb.at[b], isems[b]).wait()

    zeros16 = jnp.zeros((16,), jnp.int32)

    def fire_out(u, b):
        pltpu.async_copy(outb.at[b], out_hbm.at[pl.ds(u * 2048, 2048)],
                         osems[b])

    def wait_out(b):
        pltpu.make_async_copy(outb.at[b],
                              out_hbm.at[pl.ds(0, 2048)], osems[b]).wait()

    def transpose(b):
        for v in range(128):
            col = plsc.load_gather(
                inb.at[b], [zeros16, iota16, jnp.full((16,), v, jnp.int32)])
            outb[b, pl.ds(v * EMB, EMB)] = col

    for b in range(2):
        @pl.when(u0 + b < N_UNITS)
        def _():
            fire_in(u0 + b, b)

    def outer(g, carry):
        for b in range(2):
            u = u0 + g * 2 + b

            @pl.when(u < N_UNITS)
            def _process(b=b, u=u):
                wait_in(b)

                @pl.when(g > 0)
                def _drain():
                    wait_out(b)
                transpose(b)
                fire_out(u, b)

                @pl.when(u + 2 < N_UNITS)
                def _prefetch():
                    fire_in(u + 2, b)
        return carry

    lax.fori_loop(0, UPW // 2, outer, 0)
    for b in range(2):
        @pl.when(u0 + b < N_UNITS)
        def _():
            wait_out(b)


@functools.cache
def _sc_pack():
    mesh = plsc.VectorSubcoreMesh(
        core_axis_name="c", subcore_axis_name="s",
        num_cores=_NC, num_subcores=_NS)
    return pl.kernel(
        _sc_pack_body,
        mesh=mesh,
        out_type=jax.ShapeDtypeStruct((N_UNITS * 2048,), jnp.float32),
        scratch_types=[
            pltpu.VMEM((2, 1, EMB, 128), jnp.float32),
            pltpu.VMEM((2, 2048), jnp.float32),
            pltpu.SemaphoreType.DMA,
            pltpu.SemaphoreType.DMA,
            pltpu.SemaphoreType.DMA,
            pltpu.SemaphoreType.DMA,
        ],
        compiler_params=pltpu.CompilerParams(needs_layout_passes=False),
    )


def _sc_gather_body(gidx_hbm, table_hbm, out_hbm, idx_v, rows_v, gsem, osem):
    wid = lax.axis_index("s") * _NC + lax.axis_index("c")
    base = wid * BPW
    pltpu.sync_copy(gidx_hbm.at[:, pl.ds(base, BPW)], idx_v)

    def fire_gathers(f):
        buf = rows_v.at[f % 2]
        return [
            pltpu.async_copy(
                table_hbm.at[idx_v.at[f, pl.ds(c * CHUNK, CHUNK)]],
                buf.at[pl.ds(c * CHUNK, CHUNK)],
                gsem,
            )
            for c in range(NCHUNK)
        ]

    def fire_out(f):
        return pltpu.async_copy(
            rows_v.at[f % 2],
            out_hbm.at[pl.ds(base, BPW), pl.ds(f * EMB, EMB)],
            osem,
        )

    gathers = {}
    outs = {}
    for f in range(N_FIELDS):
        if f >= 2:
            outs[f - 2].wait()
        gathers[f] = fire_gathers(f)
        if f >= 1:
            for cp in gathers[f - 1]:
                cp.wait()
            outs[f - 1] = fire_out(f - 1)
    for cp in gathers[N_FIELDS - 1]:
        cp.wait()
    outs[N_FIELDS - 1] = fire_out(N_FIELDS - 1)
    outs[N_FIELDS - 2].wait()
    outs[N_FIELDS - 1].wait()


@functools.cache
def _sc_gather():
    mesh = plsc.VectorSubcoreMesh(
        core_axis_name="c", subcore_axis_name="s",
        num_cores=_NC, num_subcores=_NS)
    return pl.kernel(
        _sc_gather_body,
        mesh=mesh,
        out_type=jax.ShapeDtypeStruct((B, FDIM), jnp.float32),
        scratch_types=[
            pltpu.VMEM((N_FIELDS, BPW), jnp.int32),
            pltpu.VMEM((2, BPW, EMB), jnp.float32),
            pltpu.SemaphoreType.DMA,
            pltpu.SemaphoreType.DMA,
        ],
        compiler_params=pltpu.CompilerParams(use_tc_tiling_on_sc=False),
    )


BLK = 512


def _mlp_body(x_ref, a1_ref, b1_ref, lin_ref, bias_ref, w2c_ref, w2f_ref,
              w2cf_ref, b2_ref, w3c_ref, w3f_ref, w3cf_ref, out_ref):
    x = x_ref[...]
    h1 = jnp.maximum(
        jnp.dot(x, a1_ref[...], preferred_element_type=jnp.float32)
        + b1_ref[...], 0.0)
    logits = (jnp.dot(x, lin_ref[...], preferred_element_type=jnp.float32)
              + bias_ref[...])
    w2s = (w2c_ref, w2f_ref, w2cf_ref)
    w3s = (w3c_ref, w3f_ref, w3cf_ref)
    for t in range(3):
        h2 = jnp.maximum(
            jnp.dot(h1[:, t * H1:(t + 1) * H1], w2s[t][...],
                    preferred_element_type=jnp.float32)
            + b2_ref[:, t * H2:(t + 1) * H2], 0.0)
        logits = logits + jnp.dot(h2, w3s[t][...],
                                  preferred_element_type=jnp.float32)
    p = 1.0 / (1.0 + jnp.exp(-logits))
    out_ref[...] = jnp.concatenate([p, p[:, 0:1] * p[:, 1:2]], axis=1)


def _full(shape):
    return pl.BlockSpec(shape, lambda i: (0,) * len(shape))


_mlp_call = pl.pallas_call(
    _mlp_body,
    grid=(B // BLK,),
    in_specs=[
        pl.BlockSpec((BLK, FDIM), lambda i: (i, 0)),
        _full((FDIM, 3 * H1)),
        _full((1, 3 * H1)),
        _full((FDIM, 3)),
        _full((1, 3)),
        _full((H1, H2)),
        _full((H1, H2)),
        _full((H1, H2)),
        _full((1, 3 * H2)),
        _full((H2, 3)),
        _full((H2, 3)),
        _full((H2, 3)),
    ],
    out_specs=pl.BlockSpec((BLK, 4), lambda i: (i, 0)),
    out_shape=jax.ShapeDtypeStruct((B, 4), jnp.float32),
)


def kernel(indices, tables, ctr_W1, ctr_b1, ctr_W2, ctr_b2, ctr_W3, ctr_b3,
           cvrf_W1, cvrf_b1, cvrf_W2, cvrf_b2, cvrf_W3, cvrf_b3,
           cvrcf_W1, cvrcf_b1, cvrcf_W2, cvrcf_b2, cvrcf_W3, cvrcf_b3,
           w_lin_ctr, b_lin_ctr, w_lin_f, b_lin_f, w_lin_cf, b_lin_cf):
    tt = jnp.transpose(tables, (0, 2, 1))
    packed = _sc_pack()(tt)
    flat_tables = packed.reshape(N_FIELDS * VPAD, EMB)
    offs = (jnp.arange(N_FIELDS, dtype=jnp.int32) * VPAD)[:, None]
    gidx = indices.astype(jnp.int32) + offs

    feats = _sc_gather()(gidx, flat_tables)

    a1 = jnp.concatenate(
        [jnp.zeros((WIDE_DIM, 3 * H1), jnp.float32),
         jnp.concatenate([ctr_W1, cvrf_W1, cvrcf_W1], axis=1)], axis=0)
    b1 = jnp.concatenate([ctr_b1, cvrf_b1, cvrcf_b1]).reshape(1, 3 * H1)
    lin = jnp.concatenate(
        [jnp.concatenate([w_lin_ctr, w_lin_f, w_lin_cf], axis=1),
         jnp.zeros((DEEP_DIM, 3), jnp.float32)], axis=0)
    b2 = jnp.concatenate([ctr_b2, cvrf_b2, cvrcf_b2]).reshape(1, 3 * H2)
    bias = jnp.stack([ctr_b3[0] + b_lin_ctr[0], cvrf_b3[0] + b_lin_f[0],
                      cvrcf_b3[0] + b_lin_cf[0]]).reshape(1, 3)
    z1 = jnp.zeros((H2, 1), jnp.float32)
    w3c = jnp.concatenate([ctr_W3, z1, z1], axis=1)
    w3f = jnp.concatenate([z1, cvrf_W3, z1], axis=1)
    w3cf = jnp.concatenate([z1, z1, cvrcf_W3], axis=1)

    out4 = _mlp_call(feats, a1, b1, lin, bias, ctr_W2, cvrf_W2, cvrcf_W2, b2,
                     w3c, w3f, w3cf)
    return out4[:, 0:1], out4[:, 1:2], out4[:, 2:3], out4[:, 3:4]


# speedup vs baseline: 1.1307x; 1.1307x over previous
import functools

import jax
import jax.numpy as jnp
from jax import lax
from jax.experimental import pallas as pl
from jax.experimental.pallas import tpu as pltpu
from jax.experimental.pallas import tpu_sc as plsc

N_FIELDS = 26
N_WIDE = 6
B = 16384
VOCAB = 100000
EMB = 16
WIDE_DIM = N_WIDE * EMB
DEEP_DIM = (N_FIELDS - N_WIDE) * EMB
FDIM = N_FIELDS * EMB
H1, H2 = 256, 128

_NC, _NS = 2, 16
NW = _NC * _NS
BPW = B // NW
CHUNK = 128
NCHUNK = BPW // CHUNK

TCOLS = 782
VPAD = TCOLS * 128
N_UNITS = N_FIELDS * TCOLS
UPW = -(-N_UNITS // NW) // 2 * 2


def _sc_pack_body(tt_hbm, out_hbm, inb, outb, isem0, isem1, osem0, osem1):
    wid = lax.axis_index("s") * _NC + lax.axis_index("c")
    u0 = wid * UPW
    iota16 = lax.iota(jnp.int32, 16)
    isems = (isem0, isem1)
    osems = (osem0, osem1)

    def fire_in(u, b):
        f = u // TCOLS
        tc = u % TCOLS
        pltpu.async_copy(tt_hbm.at[pl.ds(f, 1), :, pl.ds(tc * 128, 128)],
                         inb.at[b], isems[b])

    def wait_in(b):
        pltpu.make_async_copy(tt_hbm.at[pl.ds(0, 1), :, pl.ds(0, 128)],
                              inb.at[b], isems[b]).wait()

    zeros16 = jnp.zeros((16,), jnp.int32)

    def fire_out(u, b):
        pltpu.async_copy(outb.at[b], out_hbm.at[pl.ds(u * 2048, 2048)],
                         osems[b])

    def wait_out(b):
        pltpu.make_async_copy(outb.at[b],
                              out_hbm.at[pl.ds(0, 2048)], osems[b]).wait()

    def transpose(b):
        def body(g, carry):
            v0 = g * 16
            for k in range(16):
                col = plsc.load_gather(
                    inb.at[b],
                    [zeros16, iota16, jnp.full((16,), k, jnp.int32) + v0])
                outb[b, pl.ds((v0 + k) * EMB, EMB)] = col
            return carry
        lax.fori_loop(0, 8, body, 0)

    for b in range(2):
        @pl.when(u0 + b < N_UNITS)
        def _():
            fire_in(u0 + b, b)

    def outer(g, carry):
        for b in range(2):
            u = u0 + g * 2 + b

            @pl.when(u < N_UNITS)
            def _process(b=b, u=u):
                wait_in(b)

                @pl.when(g > 0)
                def _drain():
                    wait_out(b)
                transpose(b)
                fire_out(u, b)

                @pl.when(u + 2 < N_UNITS)
                def _prefetch():
                    fire_in(u + 2, b)
        return carry

    lax.fori_loop(0, UPW // 2, outer, 0)
    for b in range(2):
        @pl.when(u0 + b < N_UNITS)
        def _():
            wait_out(b)


@functools.cache
def _sc_pack():
    mesh = plsc.VectorSubcoreMesh(
        core_axis_name="c", subcore_axis_name="s",
        num_cores=_NC, num_subcores=_NS)
    return pl.kernel(
        _sc_pack_body,
        mesh=mesh,
        out_type=jax.ShapeDtypeStruct((N_UNITS * 2048,), jnp.float32),
        scratch_types=[
            pltpu.VMEM((2, 1, EMB, 128), jnp.float32),
            pltpu.VMEM((2, 2048), jnp.float32),
            pltpu.SemaphoreType.DMA,
            pltpu.SemaphoreType.DMA,
            pltpu.SemaphoreType.DMA,
            pltpu.SemaphoreType.DMA,
        ],
        compiler_params=pltpu.CompilerParams(needs_layout_passes=False),
    )


def _sc_gather_body(gidx_hbm, table_hbm, out_hbm, idx_v, rows_v, gsem, osem):
    wid = lax.axis_index("s") * _NC + lax.axis_index("c")
    base = wid * BPW
    pltpu.sync_copy(gidx_hbm.at[:, pl.ds(base, BPW)], idx_v)

    def fire_gathers(f):
        buf = rows_v.at[f % 2]
        return [
            pltpu.async_copy(
                table_hbm.at[idx_v.at[f, pl.ds(c * CHUNK, CHUNK)]],
                buf.at[pl.ds(c * CHUNK, CHUNK)],
                gsem,
            )
            for c in range(NCHUNK)
        ]

    def fire_out(f):
        return pltpu.async_copy(
            rows_v.at[f % 2],
            out_hbm.at[pl.ds(base, BPW), pl.ds(f * EMB, EMB)],
            osem,
        )

    gathers = {}
    outs = {}
    for f in range(N_FIELDS):
        if f >= 2:
            outs[f - 2].wait()
        gathers[f] = fire_gathers(f)
        if f >= 1:
            for cp in gathers[f - 1]:
                cp.wait()
            outs[f - 1] = fire_out(f - 1)
    for cp in gathers[N_FIELDS - 1]:
        cp.wait()
    outs[N_FIELDS - 1] = fire_out(N_FIELDS - 1)
    outs[N_FIELDS - 2].wait()
    outs[N_FIELDS - 1].wait()


@functools.cache
def _sc_gather():
    mesh = plsc.VectorSubcoreMesh(
        core_axis_name="c", subcore_axis_name="s",
        num_cores=_NC, num_subcores=_NS)
    return pl.kernel(
        _sc_gather_body,
        mesh=mesh,
        out_type=jax.ShapeDtypeStruct((B, FDIM), jnp.float32),
        scratch_types=[
            pltpu.VMEM((N_FIELDS, BPW), jnp.int32),
            pltpu.VMEM((2, BPW, EMB), jnp.float32),
            pltpu.SemaphoreType.DMA,
            pltpu.SemaphoreType.DMA,
        ],
        compiler_params=pltpu.CompilerParams(use_tc_tiling_on_sc=False),
    )


BLK = 512


def _mlp_body(x_ref, a1_ref, b1_ref, lin_ref, bias_ref, w2c_ref, w2f_ref,
              w2cf_ref, b2_ref, w3c_ref, w3f_ref, w3cf_ref, out_ref):
    x = x_ref[...]
    h1 = jnp.maximum(
        jnp.dot(x, a1_ref[...], preferred_element_type=jnp.float32)
        + b1_ref[...], 0.0)
    logits = (jnp.dot(x, lin_ref[...], preferred_element_type=jnp.float32)
              + bias_ref[...])
    w2s = (w2c_ref, w2f_ref, w2cf_ref)
    w3s = (w3c_ref, w3f_ref, w3cf_ref)
    for t in range(3):
        h2 = jnp.maximum(
            jnp.dot(h1[:, t * H1:(t + 1) * H1], w2s[t][...],
                    preferred_element_type=jnp.float32)
            + b2_ref[:, t * H2:(t + 1) * H2], 0.0)
        logits = logits + jnp.dot(h2, w3s[t][...],
                                  preferred_element_type=jnp.float32)
    p = 1.0 / (1.0 + jnp.exp(-logits))
    out_ref[...] = jnp.concatenate([p, p[:, 0:1] * p[:, 1:2]], axis=1)


def _full(shape):
    return pl.BlockSpec(shape, lambda i: (0,) * len(shape))


_mlp_call = pl.pallas_call(
    _mlp_body,
    grid=(B // BLK,),
    in_specs=[
        pl.BlockSpec((BLK, FDIM), lambda i: (i, 0)),
        _full((FDIM, 3 * H1)),
        _full((1, 3 * H1)),
        _full((FDIM, 3)),
        _full((1, 3)),
        _full((H1, H2)),
        _full((H1, H2)),
        _full((H1, H2)),
        _full((1, 3 * H2)),
        _full((H2, 3)),
        _full((H2, 3)),
        _full((H2, 3)),
    ],
    out_specs=pl.BlockSpec((BLK, 4), lambda i: (i, 0)),
    out_shape=jax.ShapeDtypeStruct((B, 4), jnp.float32),
)


def kernel(indices, tables, ctr_W1, ctr_b1, ctr_W2, ctr_b2, ctr_W3, ctr_b3,
           cvrf_W1, cvrf_b1, cvrf_W2, cvrf_b2, cvrf_W3, cvrf_b3,
           cvrcf_W1, cvrcf_b1, cvrcf_W2, cvrcf_b2, cvrcf_W3, cvrcf_b3,
           w_lin_ctr, b_lin_ctr, w_lin_f, b_lin_f, w_lin_cf, b_lin_cf):
    tt = jnp.transpose(tables, (0, 2, 1))
    packed = _sc_pack()(tt)
    flat_tables = packed.reshape(N_FIELDS * VPAD, EMB)
    offs = (jnp.arange(N_FIELDS, dtype=jnp.int32) * VPAD)[:, None]
    gidx = indices.astype(jnp.int32) + offs

    feats = _sc_gather()(gidx, flat_tables)

    a1 = jnp.concatenate(
        [jnp.zeros((WIDE_DIM, 3 * H1), jnp.float32),
         jnp.concatenate([ctr_W1, cvrf_W1, cvrcf_W1], axis=1)], axis=0)
    b1 = jnp.concatenate([ctr_b1, cvrf_b1, cvrcf_b1]).reshape(1, 3 * H1)
    lin = jnp.concatenate(
        [jnp.concatenate([w_lin_ctr, w_lin_f, w_lin_cf], axis=1),
         jnp.zeros((DEEP_DIM, 3), jnp.float32)], axis=0)
    b2 = jnp.concatenate([ctr_b2, cvrf_b2, cvrcf_b2]).reshape(1, 3 * H2)
    bias = jnp.stack([ctr_b3[0] + b_lin_ctr[0], cvrf_b3[0] + b_lin_f[0],
                      cvrcf_b3[0] + b_lin_cf[0]]).reshape(1, 3)
    z1 = jnp.zeros((H2, 1), jnp.float32)
    w3c = jnp.concatenate([ctr_W3, z1, z1], axis=1)
    w3f = jnp.concatenate([z1, cvrf_W3, z1], axis=1)
    w3cf = jnp.concatenate([z1, z1, cvrcf_W3], axis=1)

    out4 = _mlp_call(feats, a1, b1, lin, bias, ctr_W2, cvrf_W2, cvrcf_W2, b2,
                     w3c, w3f, w3cf)
    return out4[:, 0:1], out4[:, 1:2], out4[:, 2:3], out4[:, 3:4]
